# baseline (speedup 1.0000x reference)
import functools

import jax
import jax.numpy as jnp
from jax import lax
from jax.experimental import pallas as pl
from jax.experimental.pallas import tpu as pltpu
from jax.experimental.pallas import tpu_sc as plsc

N = 10000
E = 320000
D = 128

NC = 2
NS = 16
NW = NC * NS
CH = 80
EPT = 10080
NCHUNK = EPT // CH
G = 9
NGRP = NCHUNK // G
NB = 3
LK = 2
EVP = 80
EPAD = NW * EPT - E
NPAD = 10240
RPT = NPAD // NS


def _spmm_sc(h, row3, col3, ev3):
    mesh = plsc.VectorSubcoreMesh(core_axis_name="c", subcore_axis_name="s")

    @functools.partial(
        pl.kernel,
        out_type=jax.ShapeDtypeStruct((NC, NPAD, D), jnp.float32),
        mesh=mesh,
        scratch_types=(
            [pltpu.VMEM((2, G, CH), jnp.int32)]
            + [pltpu.VMEM((2, G, CH), jnp.int32)]
            + [pltpu.VMEM((2, G, EVP), jnp.float32)]
            + [pltpu.VMEM((NB * CH, D), jnp.float32)]
            + [pltpu.VMEM_SHARED((NPAD, D), jnp.float32)]
            + [pltpu.SemaphoreType.DMA] * (2 * NB + 2)
        ),
    )
    def k(h_hbm, row_hbm, col_hbm, ev_hbm, out_hbm, *scr):
        rowv2, colv2, evv2, allbuf, acc = scr[0:5]
        gsem = scr[5:5 + NB]
        ssem = scr[5 + NB:5 + 2 * NB]
        isem = scr[5 + 2 * NB:5 + 2 * NB + 2]
        bufs = [allbuf.at[pl.ds(ln * CH, CH)] for ln in range(NB)]
        sid = lax.axis_index("s")
        cid = lax.axis_index("c")
        wid = cid * NS + sid

        def stage(gg, par, sem_op):
            sem_op(pltpu.make_async_copy(row_hbm.at[wid, gg], rowv2.at[par],
                                         isem[par]))
            sem_op(pltpu.make_async_copy(col_hbm.at[wid, gg], colv2.at[par],
                                         isem[par]))
            sem_op(pltpu.make_async_copy(ev_hbm.at[wid, gg], evv2.at[par],
                                         isem[par]))

        stage(0, 0, lambda c: c.start())
        stage(1, 1, lambda c: c.start())

        zero = jnp.zeros((16,), jnp.float32)
        nbr = NB * CH

        def zb(i, _):
            allbuf[i // 8, pl.ds((i % 8) * 16, 16)] = zero
            return 0

        lax.fori_loop(0, nbr * 8, zb, 0)
        base = sid * RPT
        pltpu.async_copy(allbuf, acc.at[pl.ds(base, nbr)], gsem[0])
        pltpu.async_copy(allbuf, acc.at[pl.ds(base + nbr, nbr)], gsem[0])
        pltpu.async_copy(allbuf.at[pl.ds(0, RPT - 2 * nbr)],
                         acc.at[pl.ds(base + 2 * nbr, RPT - 2 * nbr)], gsem[0])
        pltpu.make_async_copy(allbuf, acc.at[pl.ds(base, nbr)], gsem[0]).wait()
        pltpu.make_async_copy(allbuf, acc.at[pl.ds(base, nbr)], gsem[0]).wait()
        pltpu.make_async_copy(allbuf.at[pl.ds(0, RPT - 2 * nbr)],
                              acc.at[pl.ds(base, RPT - 2 * nbr)],
                              gsem[0]).wait()
        plsc.subcore_barrier()

        def scale(ln, par, j):
            for g in range((CH + 15) // 16):
                ne = min(16, CH - g * 16)
                evv16 = evv2[par, j, pl.ds(g * 16, 16)]
                for e in range(ne):
                    evb = evv16.at[jnp.full((16,), e, jnp.int32)].get(
                        mode="promise_in_bounds")
                    r = ln * CH + g * 16 + e
                    for q in range(8):
                        allbuf[r, pl.ds(q * 16, 16)] = (
                            allbuf[r, pl.ds(q * 16, 16)] * evb)

        def run_group(gg, par):
            stage(gg, par, lambda c: c.wait())
            for ln in range(LK):
                pltpu.async_copy(h_hbm.at[colv2.at[par, ln]], bufs[ln],
                                 gsem[ln])

            def step(p, _):
                for ln in range(NB):
                    j = NB * p + ln
                    buf = bufs[ln]
                    nxt = (ln + LK) % NB
                    pltpu.make_async_copy(h_hbm.at[colv2.at[par, j]], buf,
                                          gsem[ln]).wait()
                    scale(ln, par, j)
                    pltpu.async_copy(buf, acc.at[rowv2.at[par, j]], ssem[ln],
                                     add=True)

                    def wait_s():
                        pltpu.make_async_copy(bufs[nxt],
                                              acc.at[rowv2.at[par, 0]],
                                              ssem[nxt]).wait()

                    def issue_g():
                        pltpu.async_copy(h_hbm.at[colv2.at[par, j + LK]],
                                         bufs[nxt], gsem[nxt])

                    if ln < NB - LK:
                        @pl.when(p > 0)
                        def _():
                            wait_s()

                        issue_g()
                    else:
                        wait_s()

                        @pl.when(p < G // NB - 1)
                        def _():
                            issue_g()
                return 0

            lax.fori_loop(0, G // NB, step, 0)
            for i in range(G - (NB - LK), G):
                pltpu.make_async_copy(bufs[i % NB], acc.at[rowv2.at[par, 0]],
                                      ssem[i % NB]).wait()

        def grp_pair(q, _):
            gg0 = 2 * q
            run_group(gg0, 0)

            @pl.when(gg0 + 2 < NGRP)
            def _():
                stage(gg0 + 2, 0, lambda c: c.start())

            run_group(gg0 + 1, 1)

            @pl.when(gg0 + 3 < NGRP)
            def _():
                stage(gg0 + 3, 1, lambda c: c.start())

            return 0

        lax.fori_loop(0, NGRP // 2, grp_pair, 0)
        plsc.subcore_barrier()
        pltpu.sync_copy(acc.at[pl.ds(base, RPT)],
                        out_hbm.at[cid, pl.ds(base, RPT)])

    return k(h, row3, col3, ev3)


BM = 1000


def _mm2(x, Wa, ba, Wb, bb):
    def kfn(x_ref, wa_ref, ba_ref, wb_ref, bb_ref, a_ref, h_ref):
        xb = x_ref[...]
        a_ref[...] = jnp.dot(xb, wa_ref[...], preferred_element_type=jnp.float32) + ba_ref[...]
        h_ref[...] = jnp.dot(xb, wb_ref[...], preferred_element_type=jnp.float32) + bb_ref[...]

    return pl.pallas_call(
        kfn,
        grid=(N // BM,),
        in_specs=[
            pl.BlockSpec((BM, D), lambda i: (i, 0)),
            pl.BlockSpec((D, D), lambda i: (0, 0)),
            pl.BlockSpec((1, D), lambda i: (0, 0)),
            pl.BlockSpec((D, D), lambda i: (0, 0)),
            pl.BlockSpec((1, D), lambda i: (0, 0)),
        ],
        out_specs=[
            pl.BlockSpec((BM, D), lambda i: (i, 0)),
            pl.BlockSpec((BM, D), lambda i: (i, 0)),
        ],
        out_shape=[jax.ShapeDtypeStruct((N, D), jnp.float32)] * 2,
    )(x, Wa, ba.reshape(1, D), Wb, bb.reshape(1, D))


def _gate_mm(a0, nvec, s0, s1, W, b):
    def kfn(a0_ref, n_ref, s0_ref, s1_ref, w_ref, b_ref, h_ref):
        nb = n_ref[...]
        x1 = a0_ref[...] * nb + (s0_ref[...] + s1_ref[...]) * (1.0 - nb)
        h_ref[...] = jnp.dot(x1, w_ref[...], preferred_element_type=jnp.float32) + b_ref[...]

    return pl.pallas_call(
        kfn,
        grid=(N // BM,),
        in_specs=[
            pl.BlockSpec((BM, D), lambda i: (i, 0)),
            pl.BlockSpec((BM, 1), lambda i: (i, 0)),
            pl.BlockSpec((BM, D), lambda i: (i, 0)),
            pl.BlockSpec((BM, D), lambda i: (i, 0)),
            pl.BlockSpec((D, D), lambda i: (0, 0)),
            pl.BlockSpec((1, D), lambda i: (0, 0)),
        ],
        out_specs=pl.BlockSpec((BM, D), lambda i: (i, 0)),
        out_shape=jax.ShapeDtypeStruct((N, D), jnp.float32),
    )(a0, nvec, s0, s1, W, b.reshape(1, D))


def _add_mm(s0, s1, W, b):
    def kfn(s0_ref, s1_ref, w_ref, b_ref, h_ref):
        x = s0_ref[...] + s1_ref[...]
        h_ref[...] = jnp.dot(x, w_ref[...], preferred_element_type=jnp.float32) + b_ref[...]

    return pl.pallas_call(
        kfn,
        grid=(N // BM,),
        in_specs=[
            pl.BlockSpec((BM, D), lambda i: (i, 0)),
            pl.BlockSpec((BM, D), lambda i: (i, 0)),
            pl.BlockSpec((D, D), lambda i: (0, 0)),
            pl.BlockSpec((1, D), lambda i: (0, 0)),
        ],
        out_specs=pl.BlockSpec((BM, D), lambda i: (i, 0)),
        out_shape=jax.ShapeDtypeStruct((N, D), jnp.float32),
    )(s0, s1, W, b.reshape(1, D))


def _finalize(s0, s1, a0):
    def kfn(s0_ref, s1_ref, a0_ref, o_ref):
        o_ref[:, :D] = s0_ref[...] + s1_ref[...]
        o_ref[:, D:] = a0_ref[...]

    return pl.pallas_call(
        kfn,
        grid=(N // BM,),
        in_specs=[
            pl.BlockSpec((BM, D), lambda i: (i, 0)),
            pl.BlockSpec((BM, D), lambda i: (i, 0)),
            pl.BlockSpec((BM, D), lambda i: (i, 0)),
        ],
        out_specs=pl.BlockSpec((BM, 2 * D), lambda i: (i, 0)),
        out_shape=jax.ShapeDtypeStruct((N, 2 * D), jnp.float32),
    )(s0, s1, a0)


def kernel(A1_tensor, edge_index, edge_values, Lin1, Lin1_bias, n, W1, b1,
           W2, b2, W3, b3):
    x = A1_tensor[:, 1:]
    row3 = jnp.pad(edge_index[0], (0, EPAD)).reshape(NW, NGRP, G, CH)
    col3 = jnp.pad(edge_index[1], (0, EPAD)).reshape(NW, NGRP, G, CH)
    ev3 = jnp.pad(edge_values, (0, EPAD)).reshape(NW, NGRP, G, EVP)

    a0, h1 = _mm2(x, Lin1, Lin1_bias, W1, b1)
    s1 = _spmm_sc(h1, row3, col3, ev3)
    h2 = _gate_mm(a0, n, s1[0, :N], s1[1, :N], W2, b2)
    s2 = _spmm_sc(h2, row3, col3, ev3)
    h3 = _add_mm(s2[0, :N], s2[1, :N], W3, b3)
    s3 = _spmm_sc(h3, row3, col3, ev3)
    return _finalize(s3[0, :N], s3[1, :N], a0)


# speedup vs baseline: 1.2216x; 1.2216x over previous
import functools

import jax
import jax.numpy as jnp
from jax import lax
from jax.experimental import pallas as pl
from jax.experimental.pallas import tpu as pltpu
from jax.experimental.pallas import tpu_sc as plsc

N = 10000
E = 320000
D = 128

NC = 2
NS = 16
NW = NC * NS
CH = 50
EPT = E // NW
NCHUNK = EPT // CH
G = 20
NGRP = NCHUNK // G
NB = 4
LK = 2
EVP = 64
NPAD = 10240
RPT = NPAD // NS


def _spmm_sc(h, row3, col3, ev3):
    mesh = plsc.VectorSubcoreMesh(core_axis_name="c", subcore_axis_name="s")

    @functools.partial(
        pl.kernel,
        out_type=jax.ShapeDtypeStruct((NC, NPAD, D), jnp.float32),
        mesh=mesh,
        scratch_types=(
            [pltpu.VMEM((2, G, CH), jnp.int32)]
            + [pltpu.VMEM((2, G, CH), jnp.int32)]
            + [pltpu.VMEM((2, G, EVP), jnp.float32)]
            + [pltpu.VMEM((NB * CH, D), jnp.float32)]
            + [pltpu.VMEM_SHARED((NPAD, D), jnp.float32)]
            + [pltpu.SemaphoreType.DMA] * (2 * NB + 2)
        ),
    )
    def k(h_hbm, row_hbm, col_hbm, ev_hbm, out_hbm, *scr):
        rowv2, colv2, evv2, allbuf, acc = scr[0:5]
        gsem = scr[5:5 + NB]
        ssem = scr[5 + NB:5 + 2 * NB]
        isem = scr[5 + 2 * NB:5 + 2 * NB + 2]
        bufs = [allbuf.at[pl.ds(ln * CH, CH)] for ln in range(NB)]
        sid = lax.axis_index("s")
        cid = lax.axis_index("c")
        wid = cid * NS + sid

        def stage(gg, par, sem_op):
            sem_op(pltpu.make_async_copy(row_hbm.at[wid, gg], rowv2.at[par],
                                         isem[par]))
            sem_op(pltpu.make_async_copy(col_hbm.at[wid, gg], colv2.at[par],
                                         isem[par]))
            sem_op(pltpu.make_async_copy(ev_hbm.at[wid, gg], evv2.at[par],
                                         isem[par]))

        stage(0, 0, lambda c: c.start())
        stage(1, 1, lambda c: c.start())

        zero = jnp.zeros((16,), jnp.float32)
        nbr = NB * CH
        nzc = RPT // nbr
        rem = RPT - nzc * nbr

        def zb(i, _):
            allbuf[i // 8, pl.ds((i % 8) * 16, 16)] = zero
            return 0

        lax.fori_loop(0, nbr * 8, zb, 0)
        base = sid * RPT
        for t in range(nzc):
            pltpu.async_copy(allbuf, acc.at[pl.ds(base + t * nbr, nbr)],
                             gsem[0])
        pltpu.async_copy(allbuf.at[pl.ds(0, rem)],
                         acc.at[pl.ds(base + nzc * nbr, rem)], gsem[0])
        for t in range(nzc):
            pltpu.make_async_copy(allbuf, acc.at[pl.ds(base, nbr)],
                                  gsem[0]).wait()
        pltpu.make_async_copy(allbuf.at[pl.ds(0, rem)],
                              acc.at[pl.ds(base, rem)], gsem[0]).wait()
        plsc.subcore_barrier()

        def scale(ln, par, j):
            for g in range((CH + 15) // 16):
                ne = min(16, CH - g * 16)
                evv16 = evv2[par, j, pl.ds(g * 16, 16)]
                for e in range(ne):
                    evb = evv16.at[jnp.full((16,), e, jnp.int32)].get(
                        mode="promise_in_bounds")
                    r = ln * CH + g * 16 + e
                    for q in range(8):
                        allbuf[r, pl.ds(q * 16, 16)] = (
                            allbuf[r, pl.ds(q * 16, 16)] * evb)

        def run_group(gg, par):
            stage(gg, par, lambda c: c.wait())
            for ln in range(LK):
                pltpu.async_copy(h_hbm.at[colv2.at[par, ln]], bufs[ln],
                                 gsem[ln])

            def step(p, _):
                for ln in range(NB):
                    j = NB * p + ln
                    buf = bufs[ln]
                    nxt = (ln + LK) % NB
                    pltpu.make_async_copy(h_hbm.at[colv2.at[par, j]], buf,
                                          gsem[ln]).wait()
                    scale(ln, par, j)
                    pltpu.async_copy(buf, acc.at[rowv2.at[par, j]], ssem[ln],
                                     add=True)

                    def wait_s():
                        pltpu.make_async_copy(bufs[nxt],
                                              acc.at[rowv2.at[par, 0]],
                                              ssem[nxt]).wait()

                    def issue_g():
                        pltpu.async_copy(h_hbm.at[colv2.at[par, j + LK]],
                                         bufs[nxt], gsem[nxt])

                    if ln < NB - LK:
                        @pl.when(p > 0)
                        def _():
                            wait_s()

                        issue_g()
                    else:
                        wait_s()

                        @pl.when(p < G // NB - 1)
                        def _():
                            issue_g()
                return 0

            lax.fori_loop(0, G // NB, step, 0)
            for i in range(G - (NB - LK), G):
                pltpu.make_async_copy(bufs[i % NB], acc.at[rowv2.at[par, 0]],
                                      ssem[i % NB]).wait()

        def grp_pair(q, _):
            gg0 = 2 * q
            run_group(gg0, 0)

            @pl.when(gg0 + 2 < NGRP)
            def _():
                stage(gg0 + 2, 0, lambda c: c.start())

            run_group(gg0 + 1, 1)

            @pl.when(gg0 + 3 < NGRP)
            def _():
                stage(gg0 + 3, 1, lambda c: c.start())

            return 0

        lax.fori_loop(0, NGRP // 2, grp_pair, 0)
        plsc.subcore_barrier()
        pltpu.sync_copy(acc.at[pl.ds(base, RPT)],
                        out_hbm.at[cid, pl.ds(base, RPT)])

    return k(h, row3, col3, ev3)


BM = 1000


def _mm2(x, Wa, ba, Wb, bb):
    def kfn(x_ref, wa_ref, ba_ref, wb_ref, bb_ref, a_ref, h_ref):
        xb = x_ref[...]
        a_ref[...] = jnp.dot(xb, wa_ref[...], preferred_element_type=jnp.float32) + ba_ref[...]
        h_ref[...] = jnp.dot(xb, wb_ref[...], preferred_element_type=jnp.float32) + bb_ref[...]

    return pl.pallas_call(
        kfn,
        grid=(N // BM,),
        in_specs=[
            pl.BlockSpec((BM, D), lambda i: (i, 0)),
            pl.BlockSpec((D, D), lambda i: (0, 0)),
            pl.BlockSpec((1, D), lambda i: (0, 0)),
            pl.BlockSpec((D, D), lambda i: (0, 0)),
            pl.BlockSpec((1, D), lambda i: (0, 0)),
        ],
        out_specs=[
            pl.BlockSpec((BM, D), lambda i: (i, 0)),
            pl.BlockSpec((BM, D), lambda i: (i, 0)),
        ],
        out_shape=[jax.ShapeDtypeStruct((N, D), jnp.float32)] * 2,
    )(x, Wa, ba.reshape(1, D), Wb, bb.reshape(1, D))


def _gate_mm(a0, nvec, s, W, b):
    def kfn(a0_ref, n_ref, s0_ref, s1_ref, w_ref, b_ref, h_ref):
        nb = n_ref[...]
        x1 = a0_ref[...] * nb + (s0_ref[0] + s1_ref[0]) * (1.0 - nb)
        h_ref[...] = jnp.dot(x1, w_ref[...], preferred_element_type=jnp.float32) + b_ref[...]

    return pl.pallas_call(
        kfn,
        grid=(N // BM,),
        in_specs=[
            pl.BlockSpec((BM, D), lambda i: (i, 0)),
            pl.BlockSpec((BM, 1), lambda i: (i, 0)),
            pl.BlockSpec((1, BM, D), lambda i: (0, i, 0)),
            pl.BlockSpec((1, BM, D), lambda i: (1, i, 0)),
            pl.BlockSpec((D, D), lambda i: (0, 0)),
            pl.BlockSpec((1, D), lambda i: (0, 0)),
        ],
        out_specs=pl.BlockSpec((BM, D), lambda i: (i, 0)),
        out_shape=jax.ShapeDtypeStruct((N, D), jnp.float32),
    )(a0, nvec, s, s, W, b.reshape(1, D))


def _add_mm(s, W, b):
    def kfn(s0_ref, s1_ref, w_ref, b_ref, h_ref):
        x = s0_ref[0] + s1_ref[0]
        h_ref[...] = jnp.dot(x, w_ref[...], preferred_element_type=jnp.float32) + b_ref[...]

    return pl.pallas_call(
        kfn,
        grid=(N // BM,),
        in_specs=[
            pl.BlockSpec((1, BM, D), lambda i: (0, i, 0)),
            pl.BlockSpec((1, BM, D), lambda i: (1, i, 0)),
            pl.BlockSpec((D, D), lambda i: (0, 0)),
            pl.BlockSpec((1, D), lambda i: (0, 0)),
        ],
        out_specs=pl.BlockSpec((BM, D), lambda i: (i, 0)),
        out_shape=jax.ShapeDtypeStruct((N, D), jnp.float32),
    )(s, s, W, b.reshape(1, D))


def _finalize(s, a0):
    def kfn(s0_ref, s1_ref, a0_ref, o_ref):
        o_ref[:, :D] = s0_ref[0] + s1_ref[0]
        o_ref[:, D:] = a0_ref[...]

    return pl.pallas_call(
        kfn,
        grid=(N // BM,),
        in_specs=[
            pl.BlockSpec((1, BM, D), lambda i: (0, i, 0)),
            pl.BlockSpec((1, BM, D), lambda i: (1, i, 0)),
            pl.BlockSpec((BM, D), lambda i: (i, 0)),
        ],
        out_specs=pl.BlockSpec((BM, 2 * D), lambda i: (i, 0)),
        out_shape=jax.ShapeDtypeStruct((N, 2 * D), jnp.float32),
    )(s, s, a0)


def kernel(A1_tensor, edge_index, edge_values, Lin1, Lin1_bias, n, W1, b1,
           W2, b2, W3, b3):
    x = A1_tensor[:, 1:]
    row3 = edge_index[0].reshape(NW, NGRP, G, CH)
    col3 = edge_index[1].reshape(NW, NGRP, G, CH)
    ev3 = jnp.pad(edge_values.reshape(NW, NGRP, G, CH),
                  ((0, 0), (0, 0), (0, 0), (0, EVP - CH)))

    a0, h1 = _mm2(x, Lin1, Lin1_bias, W1, b1)
    s1 = _spmm_sc(h1, row3, col3, ev3)
    h2 = _gate_mm(a0, n, s1, W2, b2)
    s2 = _spmm_sc(h2, row3, col3, ev3)
    h3 = _add_mm(s2, W3, b3)
    s3 = _spmm_sc(h3, row3, col3, ev3)
    return _finalize(s3, a0)


# speedup vs baseline: 1.2264x; 1.0039x over previous
import functools

import jax
import jax.numpy as jnp
from jax import lax
from jax.experimental import pallas as pl
from jax.experimental.pallas import tpu as pltpu
from jax.experimental.pallas import tpu_sc as plsc

N = 10000
E = 320000
D = 128

NC = 2
NS = 16
NW = NC * NS
CH = 50
EPT = E // NW
NCHUNK = EPT // CH
G = 20
NGRP = NCHUNK // G
NB = 4
LK = 2
EVP = 64
NPAD = 10240
RPT = NPAD // NS


def _spmm_sc(h, row3, col3, ev3):
    mesh = plsc.VectorSubcoreMesh(core_axis_name="c", subcore_axis_name="s")

    @functools.partial(
        pl.kernel,
        out_type=jax.ShapeDtypeStruct((NC, NPAD, D), jnp.float32),
        mesh=mesh,
        scratch_types=(
            [pltpu.VMEM((G, CH), jnp.int32)] * 2
            + [pltpu.VMEM((G, CH), jnp.int32)] * 2
            + [pltpu.VMEM((G, EVP), jnp.float32)] * 2
            + [pltpu.VMEM((CH, D), jnp.float32)] * NB
            + [pltpu.VMEM_SHARED((NPAD, D), jnp.float32)]
            + [pltpu.SemaphoreType.DMA] * (2 * NB + 2)
        ),
    )
    def k(h_hbm, row_hbm, col_hbm, ev_hbm, out_hbm, *scr):
        rowv = scr[0:2]
        colv = scr[2:4]
        evv = scr[4:6]
        bufs = list(scr[6:6 + NB])
        acc = scr[6 + NB]
        gsem = scr[7 + NB:7 + 2 * NB]
        ssem = scr[7 + 2 * NB:7 + 3 * NB]
        isem = scr[7 + 3 * NB:7 + 3 * NB + 2]
        sid = lax.axis_index("s")
        cid = lax.axis_index("c")
        wid = cid * NS + sid

        def stage(gg, par, sem_op):
            sem_op(pltpu.make_async_copy(row_hbm.at[wid, gg], rowv[par],
                                         isem[par]))
            sem_op(pltpu.make_async_copy(col_hbm.at[wid, gg], colv[par],
                                         isem[par]))
            sem_op(pltpu.make_async_copy(ev_hbm.at[wid, gg], evv[par],
                                         isem[par]))

        stage(0, 0, lambda c: c.start())
        stage(1, 1, lambda c: c.start())

        zero = jnp.zeros((16,), jnp.float32)
        nzc = RPT // CH
        rem = RPT - nzc * CH

        def zb(i, _):
            bufs[0][i // 8, pl.ds((i % 8) * 16, 16)] = zero
            return 0

        lax.fori_loop(0, CH * 8, zb, 0)
        base = sid * RPT
        for t in range(nzc):
            pltpu.async_copy(bufs[0], acc.at[pl.ds(base + t * CH, CH)],
                             gsem[0])
        pltpu.async_copy(bufs[0].at[pl.ds(0, rem)],
                         acc.at[pl.ds(base + nzc * CH, rem)], gsem[0])
        for t in range(nzc):
            pltpu.make_async_copy(bufs[0], acc.at[pl.ds(base, CH)],
                                  gsem[0]).wait()
        pltpu.make_async_copy(bufs[0].at[pl.ds(0, rem)],
                              acc.at[pl.ds(base, rem)], gsem[0]).wait()
        plsc.subcore_barrier()

        def scale(ln, par, j):
            buf = bufs[ln]
            for g in range((CH + 15) // 16):
                ne = min(16, CH - g * 16)
                evv16 = evv[par][j, pl.ds(g * 16, 16)]
                for e in range(ne):
                    evb = evv16.at[jnp.full((16,), e, jnp.int32)].get(
                        mode="promise_in_bounds")
                    r = g * 16 + e
                    for q in range(8):
                        buf[r, pl.ds(q * 16, 16)] = (
                            buf[r, pl.ds(q * 16, 16)] * evb)

        def run_group(gg, par):
            stage(gg, par, lambda c: c.wait())
            for ln in range(LK):
                pltpu.async_copy(h_hbm.at[colv[par].at[ln]], bufs[ln],
                                 gsem[ln])

            def step(p, _):
                for ln in range(NB):
                    j = NB * p + ln
                    buf = bufs[ln]
                    nxt = (ln + LK) % NB
                    pltpu.make_async_copy(h_hbm.at[colv[par].at[j]], buf,
                                          gsem[ln]).wait()
                    scale(ln, par, j)
                    pltpu.async_copy(buf, acc.at[rowv[par].at[j]], ssem[ln],
                                     add=True)

                    def wait_s():
                        pltpu.make_async_copy(bufs[nxt],
                                              acc.at[rowv[par].at[0]],
                                              ssem[nxt]).wait()

                    def issue_g():
                        pltpu.async_copy(h_hbm.at[colv[par].at[j + LK]],
                                         bufs[nxt], gsem[nxt])

                    if ln < NB - LK:
                        @pl.when(p > 0)
                        def _():
                            wait_s()

                        issue_g()
                    else:
                        wait_s()

                        @pl.when(p < G // NB - 1)
                        def _():
                            issue_g()
                return 0

            lax.fori_loop(0, G // NB, step, 0)
            for i in range(G - (NB - LK), G):
                pltpu.make_async_copy(bufs[i % NB], acc.at[rowv[par].at[0]],
                                      ssem[i % NB]).wait()

        def grp_pair(q, _):
            gg0 = 2 * q
            run_group(gg0, 0)

            @pl.when(gg0 + 2 < NGRP)
            def _():
                stage(gg0 + 2, 0, lambda c: c.start())

            run_group(gg0 + 1, 1)

            @pl.when(gg0 + 3 < NGRP)
            def _():
                stage(gg0 + 3, 1, lambda c: c.start())

            return 0

        lax.fori_loop(0, NGRP // 2, grp_pair, 0)
        plsc.subcore_barrier()
        pltpu.sync_copy(acc.at[pl.ds(base, RPT)],
                        out_hbm.at[cid, pl.ds(base, RPT)])

    return k(h, row3, col3, ev3)


BM = 1000


def _mm2(x, Wa, ba, Wb, bb):
    def kfn(x_ref, wa_ref, ba_ref, wb_ref, bb_ref, a_ref, h_ref):
        xb = x_ref[...]
        a_ref[...] = jnp.dot(xb, wa_ref[...], preferred_element_type=jnp.float32) + ba_ref[...]
        h_ref[...] = jnp.dot(xb, wb_ref[...], preferred_element_type=jnp.float32) + bb_ref[...]

    return pl.pallas_call(
        kfn,
        grid=(N // BM,),
        in_specs=[
            pl.BlockSpec((BM, D), lambda i: (i, 0)),
            pl.BlockSpec((D, D), lambda i: (0, 0)),
            pl.BlockSpec((1, D), lambda i: (0, 0)),
            pl.BlockSpec((D, D), lambda i: (0, 0)),
            pl.BlockSpec((1, D), lambda i: (0, 0)),
        ],
        out_specs=[
            pl.BlockSpec((BM, D), lambda i: (i, 0)),
            pl.BlockSpec((BM, D), lambda i: (i, 0)),
        ],
        out_shape=[jax.ShapeDtypeStruct((N, D), jnp.float32)] * 2,
    )(x, Wa, ba.reshape(1, D), Wb, bb.reshape(1, D))


def _gate_mm(a0, nvec, s, W, b):
    def kfn(a0_ref, n_ref, s0_ref, s1_ref, w_ref, b_ref, h_ref):
        nb = n_ref[...]
        x1 = a0_ref[...] * nb + (s0_ref[0] + s1_ref[0]) * (1.0 - nb)
        h_ref[...] = jnp.dot(x1, w_ref[...], preferred_element_type=jnp.float32) + b_ref[...]

    return pl.pallas_call(
        kfn,
        grid=(N // BM,),
        in_specs=[
            pl.BlockSpec((BM, D), lambda i: (i, 0)),
            pl.BlockSpec((BM, 1), lambda i: (i, 0)),
            pl.BlockSpec((1, BM, D), lambda i: (0, i, 0)),
            pl.BlockSpec((1, BM, D), lambda i: (1, i, 0)),
            pl.BlockSpec((D, D), lambda i: (0, 0)),
            pl.BlockSpec((1, D), lambda i: (0, 0)),
        ],
        out_specs=pl.BlockSpec((BM, D), lambda i: (i, 0)),
        out_shape=jax.ShapeDtypeStruct((N, D), jnp.float32),
    )(a0, nvec, s, s, W, b.reshape(1, D))


def _add_mm(s, W, b):
    def kfn(s0_ref, s1_ref, w_ref, b_ref, h_ref):
        x = s0_ref[0] + s1_ref[0]
        h_ref[...] = jnp.dot(x, w_ref[...], preferred_element_type=jnp.float32) + b_ref[...]

    return pl.pallas_call(
        kfn,
        grid=(N // BM,),
        in_specs=[
            pl.BlockSpec((1, BM, D), lambda i: (0, i, 0)),
            pl.BlockSpec((1, BM, D), lambda i: (1, i, 0)),
            pl.BlockSpec((D, D), lambda i: (0, 0)),
            pl.BlockSpec((1, D), lambda i: (0, 0)),
        ],
        out_specs=pl.BlockSpec((BM, D), lambda i: (i, 0)),
        out_shape=jax.ShapeDtypeStruct((N, D), jnp.float32),
    )(s, s, W, b.reshape(1, D))


def _finalize(s, a0):
    def kfn(s0_ref, s1_ref, a0_ref, o_ref):
        o_ref[:, :D] = s0_ref[0] + s1_ref[0]
        o_ref[:, D:] = a0_ref[...]

    return pl.pallas_call(
        kfn,
        grid=(N // BM,),
        in_specs=[
            pl.BlockSpec((1, BM, D), lambda i: (0, i, 0)),
            pl.BlockSpec((1, BM, D), lambda i: (1, i, 0)),
            pl.BlockSpec((BM, D), lambda i: (i, 0)),
        ],
        out_specs=pl.BlockSpec((BM, 2 * D), lambda i: (i, 0)),
        out_shape=jax.ShapeDtypeStruct((N, 2 * D), jnp.float32),
    )(s, s, a0)


def kernel(A1_tensor, edge_index, edge_values, Lin1, Lin1_bias, n, W1, b1,
           W2, b2, W3, b3):
    x = A1_tensor[:, 1:]
    row3 = edge_index[0].reshape(NW, NGRP, G, CH)
    col3 = edge_index[1].reshape(NW, NGRP, G, CH)
    ev3 = jnp.pad(edge_values.reshape(NW, NGRP, G, CH),
                  ((0, 0), (0, 0), (0, 0), (0, EVP - CH)))

    a0, h1 = _mm2(x, Lin1, Lin1_bias, W1, b1)
    s1 = _spmm_sc(h1, row3, col3, ev3)
    h2 = _gate_mm(a0, n, s1, W2, b2)
    s2 = _spmm_sc(h2, row3, col3, ev3)
    h3 = _add_mm(s2, W3, b3)
    s3 = _spmm_sc(h3, row3, col3, ev3)
    return _finalize(s3, a0)


# speedup vs baseline: 1.3889x; 1.1325x over previous
import functools

import jax
import jax.numpy as jnp
from jax import lax
from jax.experimental import pallas as pl
from jax.experimental.pallas import tpu as pltpu
from jax.experimental.pallas import tpu_sc as plsc

N = 10000
E = 320000
D = 128

NC = 2
NS = 16
NW = NC * NS
CH = 50
EPT = E // NW
NCHUNK = EPT // CH
G = 40
NGRP = NCHUNK // G
NB = 4
LK = 2
EVP = 64
NPAD = 10240
RPT = NPAD // NS


def _spmm_sc(h, row3, col3, ev3):
    mesh = plsc.VectorSubcoreMesh(core_axis_name="c", subcore_axis_name="s")

    @functools.partial(
        pl.kernel,
        out_type=jax.ShapeDtypeStruct((NC, NPAD, D), jnp.float32),
        mesh=mesh,
        scratch_types=(
            [pltpu.VMEM((G, CH), jnp.int32)]
            + [pltpu.VMEM((G, CH), jnp.int32)]
            + [pltpu.VMEM((G, EVP), jnp.float32)]
            + [pltpu.VMEM((CH, D), jnp.float32)] * NB
            + [pltpu.VMEM_SHARED((NPAD, D), jnp.float32)]
            + [pltpu.SemaphoreType.DMA] * (2 * NB)
        ),
    )
    def k(h_hbm, row_hbm, col_hbm, ev_hbm, out_hbm, *scr):
        rowv = [scr[0]] * 2
        colv = [scr[1]] * 2
        evv = [scr[2]] * 2
        bufs = list(scr[3:3 + NB])
        acc = scr[3 + NB]
        gsem = scr[4 + NB:4 + 2 * NB]
        ssem = scr[4 + 2 * NB:4 + 3 * NB]
        sid = lax.axis_index("s")
        cid = lax.axis_index("c")
        wid = cid * NS + sid

        def stage(gg, par):
            pltpu.sync_copy(row_hbm.at[wid, gg], rowv[par])
            pltpu.sync_copy(col_hbm.at[wid, gg], colv[par])
            pltpu.sync_copy(ev_hbm.at[wid, gg], evv[par])

        zero = jnp.zeros((16,), jnp.float32)
        nzc = RPT // CH
        rem = RPT - nzc * CH

        def zb(i, _):
            bufs[0][i // 8, pl.ds((i % 8) * 16, 16)] = zero
            return 0

        lax.fori_loop(0, CH * 8, zb, 0)
        base = sid * RPT
        for t in range(nzc):
            pltpu.async_copy(bufs[0], acc.at[pl.ds(base + t * CH, CH)],
                             gsem[0])
        pltpu.async_copy(bufs[0].at[pl.ds(0, rem)],
                         acc.at[pl.ds(base + nzc * CH, rem)], gsem[0])
        for t in range(nzc):
            pltpu.make_async_copy(bufs[0], acc.at[pl.ds(base, CH)],
                                  gsem[0]).wait()
        pltpu.make_async_copy(bufs[0].at[pl.ds(0, rem)],
                              acc.at[pl.ds(base, rem)], gsem[0]).wait()
        plsc.subcore_barrier()

        def scale(ln, par, j):
            buf = bufs[ln]
            for g in range((CH + 15) // 16):
                ne = min(16, CH - g * 16)
                evv16 = evv[par][j, pl.ds(g * 16, 16)]
                for e in range(ne):
                    evb = evv16.at[jnp.full((16,), e, jnp.int32)].get(
                        mode="promise_in_bounds")
                    r = g * 16 + e
                    for q in range(8):
                        buf[r, pl.ds(q * 16, 16)] = (
                            buf[r, pl.ds(q * 16, 16)] * evb)

        def run_group(gg, par):
            stage(gg, par)
            for ln in range(LK):
                pltpu.async_copy(h_hbm.at[colv[par].at[ln]], bufs[ln],
                                 gsem[ln])

            def step(p, _):
                for ln in range(NB):
                    j = NB * p + ln
                    buf = bufs[ln]
                    nxt = (ln + LK) % NB
                    pltpu.make_async_copy(h_hbm.at[colv[par].at[j]], buf,
                                          gsem[ln]).wait()
                    scale(ln, par, j)
                    pltpu.async_copy(buf, acc.at[rowv[par].at[j]], ssem[ln],
                                     add=True)

                    def wait_s():
                        pltpu.make_async_copy(bufs[nxt],
                                              acc.at[rowv[par].at[0]],
                                              ssem[nxt]).wait()

                    def issue_g():
                        pltpu.async_copy(h_hbm.at[colv[par].at[j + LK]],
                                         bufs[nxt], gsem[nxt])

                    if ln < NB - LK:
                        @pl.when(p > 0)
                        def _():
                            wait_s()

                        issue_g()
                    else:
                        wait_s()

                        @pl.when(p < G // NB - 1)
                        def _():
                            issue_g()
                return 0

            lax.fori_loop(0, G // NB, step, 0)
            for i in range(G - (NB - LK), G):
                pltpu.make_async_copy(bufs[i % NB], acc.at[rowv[par].at[0]],
                                      ssem[i % NB]).wait()

        def grp(gg, _):
            run_group(gg, 0)
            return 0

        lax.fori_loop(0, NGRP, grp, 0)
        plsc.subcore_barrier()
        pltpu.sync_copy(acc.at[pl.ds(base, RPT)],
                        out_hbm.at[cid, pl.ds(base, RPT)])

    return k(h, row3, col3, ev3)


BM = 1000


def _mm2(x, Wa, ba, Wb, bb):
    def kfn(x_ref, wa_ref, ba_ref, wb_ref, bb_ref, a_ref, h_ref):
        xb = x_ref[...]
        a_ref[...] = jnp.dot(xb, wa_ref[...], preferred_element_type=jnp.float32) + ba_ref[...]
        h_ref[...] = jnp.dot(xb, wb_ref[...], preferred_element_type=jnp.float32) + bb_ref[...]

    return pl.pallas_call(
        kfn,
        grid=(N // BM,),
        in_specs=[
            pl.BlockSpec((BM, D), lambda i: (i, 0)),
            pl.BlockSpec((D, D), lambda i: (0, 0)),
            pl.BlockSpec((1, D), lambda i: (0, 0)),
            pl.BlockSpec((D, D), lambda i: (0, 0)),
            pl.BlockSpec((1, D), lambda i: (0, 0)),
        ],
        out_specs=[
            pl.BlockSpec((BM, D), lambda i: (i, 0)),
            pl.BlockSpec((BM, D), lambda i: (i, 0)),
        ],
        out_shape=[jax.ShapeDtypeStruct((N, D), jnp.float32)] * 2,
    )(x, Wa, ba.reshape(1, D), Wb, bb.reshape(1, D))


def _gate_mm(a0, nvec, s, W, b):
    def kfn(a0_ref, n_ref, s0_ref, s1_ref, w_ref, b_ref, h_ref):
        nb = n_ref[...]
        x1 = a0_ref[...] * nb + (s0_ref[0] + s1_ref[0]) * (1.0 - nb)
        h_ref[...] = jnp.dot(x1, w_ref[...], preferred_element_type=jnp.float32) + b_ref[...]

    return pl.pallas_call(
        kfn,
        grid=(N // BM,),
        in_specs=[
            pl.BlockSpec((BM, D), lambda i: (i, 0)),
            pl.BlockSpec((BM, 1), lambda i: (i, 0)),
            pl.BlockSpec((1, BM, D), lambda i: (0, i, 0)),
            pl.BlockSpec((1, BM, D), lambda i: (1, i, 0)),
            pl.BlockSpec((D, D), lambda i: (0, 0)),
            pl.BlockSpec((1, D), lambda i: (0, 0)),
        ],
        out_specs=pl.BlockSpec((BM, D), lambda i: (i, 0)),
        out_shape=jax.ShapeDtypeStruct((N, D), jnp.float32),
    )(a0, nvec, s, s, W, b.reshape(1, D))


def _add_mm(s, W, b):
    def kfn(s0_ref, s1_ref, w_ref, b_ref, h_ref):
        x = s0_ref[0] + s1_ref[0]
        h_ref[...] = jnp.dot(x, w_ref[...], preferred_element_type=jnp.float32) + b_ref[...]

    return pl.pallas_call(
        kfn,
        grid=(N // BM,),
        in_specs=[
            pl.BlockSpec((1, BM, D), lambda i: (0, i, 0)),
            pl.BlockSpec((1, BM, D), lambda i: (1, i, 0)),
            pl.BlockSpec((D, D), lambda i: (0, 0)),
            pl.BlockSpec((1, D), lambda i: (0, 0)),
        ],
        out_specs=pl.BlockSpec((BM, D), lambda i: (i, 0)),
        out_shape=jax.ShapeDtypeStruct((N, D), jnp.float32),
    )(s, s, W, b.reshape(1, D))


def _finalize(s, a0):
    def kfn(s0_ref, s1_ref, a0_ref, o_ref):
        o_ref[:, :D] = s0_ref[0] + s1_ref[0]
        o_ref[:, D:] = a0_ref[...]

    return pl.pallas_call(
        kfn,
        grid=(N // BM,),
        in_specs=[
            pl.BlockSpec((1, BM, D), lambda i: (0, i, 0)),
            pl.BlockSpec((1, BM, D), lambda i: (1, i, 0)),
            pl.BlockSpec((BM, D), lambda i: (i, 0)),
        ],
        out_specs=pl.BlockSpec((BM, 2 * D), lambda i: (i, 0)),
        out_shape=jax.ShapeDtypeStruct((N, 2 * D), jnp.float32),
    )(s, s, a0)


def kernel(A1_tensor, edge_index, edge_values, Lin1, Lin1_bias, n, W1, b1,
           W2, b2, W3, b3):
    x = A1_tensor[:, 1:]
    row3 = edge_index[0].reshape(NW, NGRP, G, CH)
    col3 = edge_index[1].reshape(NW, NGRP, G, CH)
    ev3 = jnp.pad(edge_values.reshape(NW, NGRP, G, CH),
                  ((0, 0), (0, 0), (0, 0), (0, EVP - CH)))

    a0, h1 = _mm2(x, Lin1, Lin1_bias, W1, b1)
    s1 = _spmm_sc(h1, row3, col3, ev3)
    h2 = _gate_mm(a0, n, s1, W2, b2)
    s2 = _spmm_sc(h2, row3, col3, ev3)
    h3 = _add_mm(s2, W3, b3)
    s3 = _spmm_sc(h3, row3, col3, ev3)
    return _finalize(s3, a0)
